# baseline (speedup 1.0000x reference)
import functools

import jax
import jax.numpy as jnp
from jax import lax
from jax.experimental import pallas as pl
from jax.experimental.pallas import tpu as pltpu
from jax.experimental.pallas import tpu_sc as plsc

NUM_POS = 28 * 28
EMB = 768
XD = 1536
BATCH = 64

NW = 32
B_PER_W = BATCH // NW
CHUNK = 16
NCHUNK = NUM_POS // CHUNK
LANES = 16
NVEC = EMB // LANES
NPAIR = (NCHUNK - 1) // 2


def _body(x_hbm, table_hbm, out_hbm,
          ina0, inb0, t0, outa0, outb0,
          ina1, inb1, t1, outa1, outb1,
          sem_l0, sem_l1, sem_s0, sem_s1, sem_c):
    wid = lax.axis_index("s") * 2 + lax.axis_index("c")
    ba = wid * B_PER_W
    bb = ba + 1
    slots = ((ina0, inb0, t0, outa0, outb0, sem_l0, sem_s0),
             (ina1, inb1, t1, outa1, outb1, sem_l1, sem_s1))

    def load_descs(slot, c):
        ina, inb, t = slot[0], slot[1], slot[2]
        sem = slot[5]
        p0 = c * CHUNK
        return (
            pltpu.make_async_copy(
                x_hbm.at[ba, pl.ds(p0, CHUNK), pl.ds(EMB, EMB)], ina, sem),
            pltpu.make_async_copy(
                x_hbm.at[bb, pl.ds(p0, CHUNK), pl.ds(EMB, EMB)], inb, sem),
            pltpu.make_async_copy(table_hbm.at[pl.ds(p0, CHUNK)], t, sem),
        )

    def store_descs(slot, c):
        outa, outb = slot[3], slot[4]
        sem = slot[6]
        p0 = c * CHUNK
        return (
            pltpu.make_async_copy(
                outa, out_hbm.at[ba, pl.ds(p0, CHUNK), pl.ds(EMB, EMB)], sem),
            pltpu.make_async_copy(
                outb, out_hbm.at[bb, pl.ds(p0, CHUNK), pl.ds(EMB, EMB)], sem),
        )

    def start(descs):
        for d in descs:
            d.start()

    def wait(descs):
        for d in descs:
            d.wait()

    def compute(slot):
        ina, inb, t, outa, outb = slot[:5]

        def row(r, _):
            for j in range(NVEC):
                sl = pl.ds(j * LANES, LANES)
                tv = t[r, sl]
                outa[r, sl] = ina[r, sl] + tv
                outb[r, sl] = inb[r, sl] + tv
            return 0

        lax.fori_loop(0, CHUNK, row, 0)

    first_a = pltpu.make_async_copy(
        x_hbm.at[ba, :, pl.ds(0, EMB)], out_hbm.at[ba, :, pl.ds(0, EMB)], sem_c)
    first_b = pltpu.make_async_copy(
        x_hbm.at[bb, :, pl.ds(0, EMB)], out_hbm.at[bb, :, pl.ds(0, EMB)], sem_c)
    first_a.start()
    first_b.start()

    start(load_descs(slots[0], 0))
    start(load_descs(slots[1], 1))

    def step(i, _):
        for s in (0, 1):
            c = 2 * i + s
            slot = slots[s]
            wait(load_descs(slot, c))

            @pl.when(i >= 1)
            def _():
                wait(store_descs(slot, c - 2))

            compute(slot)
            start(store_descs(slot, c))
            if s == 0:
                start(load_descs(slot, c + 2))
            else:
                @pl.when(i < NPAIR - 1)
                def _():
                    start(load_descs(slot, c + 2))
        return 0

    lax.fori_loop(0, NPAIR, step, 0)

    c_last = NCHUNK - 1
    slot = slots[0]
    wait(load_descs(slot, c_last))
    wait(store_descs(slot, c_last - 2))
    compute(slot)
    start(store_descs(slot, c_last))
    wait(store_descs(slots[1], c_last - 1))
    wait(store_descs(slot, c_last))
    first_a.wait()
    first_b.wait()


@jax.jit
def _sc_add(x, table):
    mesh = plsc.VectorSubcoreMesh(core_axis_name="c", subcore_axis_name="s")
    buf = lambda: pltpu.VMEM((CHUNK, EMB), jnp.float32)
    f = functools.partial(
        pl.kernel,
        mesh=mesh,
        out_type=jax.ShapeDtypeStruct((BATCH, NUM_POS, XD), jnp.float32),
        scratch_types=[buf() for _ in range(10)] + [
            pltpu.SemaphoreType.DMA for _ in range(5)],
    )(_body)
    return f(x, table)


def kernel(x, table):
    return _sc_add(x, table)


# speedup vs baseline: 16.9015x; 16.9015x over previous
import functools

import jax
import jax.numpy as jnp
from jax import lax
from jax.experimental import pallas as pl
from jax.experimental.pallas import tpu as pltpu
from jax.experimental.pallas import tpu_sc as plsc

NUM_POS = 28 * 28
EMB = 768
XD = 1536
BATCH = 64

NW = 32
B_PER_W = BATCH // NW
CHUNK = 8
NCHUNK = NUM_POS // CHUNK
LANES = 16
NVEC = EMB // LANES
NBUF = 4
NSTEP = NCHUNK // NBUF


def _body(x_hbm, table_hbm, out_hbm, *refs):
    xbufs = refs[0:NBUF]
    tbufs = refs[NBUF:2 * NBUF]
    lsems = refs[2 * NBUF:3 * NBUF]
    ssems = refs[3 * NBUF:4 * NBUF]
    wid = lax.axis_index("s") * 2 + lax.axis_index("c")
    ba = wid * B_PER_W
    bb = ba + 1

    def load_descs(s, c):
        p0 = c * CHUNK
        return (
            pltpu.make_async_copy(
                x_hbm.at[ba, pl.ds(p0, CHUNK)],
                xbufs[s].at[pl.ds(0, CHUNK)], lsems[s]),
            pltpu.make_async_copy(
                x_hbm.at[bb, pl.ds(p0, CHUNK)],
                xbufs[s].at[pl.ds(CHUNK, CHUNK)], lsems[s]),
            pltpu.make_async_copy(table_hbm.at[pl.ds(p0, CHUNK)],
                                  tbufs[s], lsems[s]),
        )

    def store_descs(s, c):
        p0 = c * CHUNK
        return (
            pltpu.make_async_copy(
                xbufs[s].at[pl.ds(0, CHUNK)],
                out_hbm.at[ba, pl.ds(p0, CHUNK)], ssems[s]),
            pltpu.make_async_copy(
                xbufs[s].at[pl.ds(CHUNK, CHUNK)],
                out_hbm.at[bb, pl.ds(p0, CHUNK)], ssems[s]),
        )

    def start(descs):
        for d in descs:
            d.start()

    def wait(descs):
        for d in descs:
            d.wait()

    def compute(s):
        xb, tb = xbufs[s], tbufs[s]

        def row(r, _):
            for j in range(NVEC):
                sl = pl.ds(j * LANES, LANES)
                sx = pl.ds(EMB + j * LANES, LANES)
                tv = tb[r, sl]
                xb[r, sx] = xb[r, sx] + tv
                xb[CHUNK + r, sx] = xb[CHUNK + r, sx] + tv
            return 0

        lax.fori_loop(0, CHUNK, row, 0)

    start(load_descs(0, 0))
    start(load_descs(1, 1))

    def step(i, _):
        for s in range(NBUF):
            c = NBUF * i + s
            wait(load_descs(s, c))
            compute(s)
            start(store_descs(s, c))
            s2 = (s + 2) % NBUF
            if s < 2:
                @pl.when(i >= 1)
                def _():
                    wait(store_descs(s2, c - 2))
            else:
                wait(store_descs(s2, c - 2))
            start(load_descs(s2, c + 2))
        return 0

    lax.fori_loop(0, NSTEP, step, 0)

    last = NSTEP * NBUF
    for s in range(2):
        c = last + s
        wait(load_descs(s, c))
        compute(s)
        start(store_descs(s, c))
    wait(store_descs(2, last - 2))
    wait(store_descs(3, last - 1))
    wait(store_descs(0, last))
    wait(store_descs(1, last + 1))


@jax.jit
def _sc_add(x, table):
    mesh = plsc.VectorSubcoreMesh(core_axis_name="c", subcore_axis_name="s")
    f = functools.partial(
        pl.kernel,
        mesh=mesh,
        out_type=jax.ShapeDtypeStruct((BATCH, NUM_POS, XD), jnp.float32),
        scratch_types=(
            [pltpu.VMEM((2 * CHUNK, XD), jnp.float32) for _ in range(NBUF)]
            + [pltpu.VMEM((CHUNK, EMB), jnp.float32) for _ in range(NBUF)]
            + [pltpu.SemaphoreType.DMA for _ in range(2 * NBUF)]),
    )(_body)
    return f(x, table)


def kernel(x, table):
    return _sc_add(x, table)
